# Initial kernel scaffold; baseline (speedup 1.0000x reference)
import jax
import jax.numpy as jnp
from jax.experimental import pallas as pl


def kernel(feature, pred, unlabeled_index, centroids):
    raise NotImplementedError("write your pallas kernel here")


# speedup vs baseline: 3.8720x; 3.8720x over previous
import functools

import jax
import jax.numpy as jnp
from jax.experimental import pallas as pl
from jax.experimental.pallas import tpu as pltpu

_B = 16384
_D = 64
_K = 1000
_KP = 1024
_BS = 1024
_NB = _B // _BS

_INV_SQRT_2PI = 0.3989422804014327


def _cluster_kernel(feature_ref, cnt_ref, label_ref, weight_ref, cosf_ref):
    eps = 1e-8
    cnt = cnt_ref[...]
    cnorm = jnp.sqrt(jnp.sum(cnt * cnt, axis=0, keepdims=True))
    cnn = cnt / jnp.maximum(cnorm, eps)

    col_iota = jax.lax.broadcasted_iota(jnp.int32, (_BS, _KP), 1)
    valid = col_iota < _K

    def phase1(j, carry):
        counts, sums = carry
        f = feature_ref[pl.ds(j * _BS, _BS), :]
        fnorm = jnp.sqrt(jnp.sum(f * f, axis=1, keepdims=True))
        fn = f / jnp.maximum(fnorm, eps)
        cos = jnp.dot(fn, cnn, preferred_element_type=jnp.float32)
        cos = jnp.where(valid, cos, -2.0)
        cos_f = jnp.max(cos, axis=1, keepdims=True)
        lab = jnp.min(jnp.where(cos == cos_f, col_iota, _KP), axis=1,
                      keepdims=True)
        onehot = (col_iota == lab).astype(jnp.float32)
        counts = counts + jnp.sum(onehot, axis=0, keepdims=True)
        sums = sums + jnp.sum(onehot * cos_f, axis=0, keepdims=True)
        cosf_ref[pl.ds(j * _BS, _BS), :] = cos_f
        label_ref[pl.ds(j * _BS, _BS), :] = lab.astype(jnp.float32)
        return counts, sums

    zero_row = jnp.zeros((1, _KP), jnp.float32)
    counts, sums = jax.lax.fori_loop(0, _NB, phase1, (zero_row, zero_row))
    mean = sums / jnp.maximum(counts, 1.0)

    def phase2(j, sq):
        cos_f = cosf_ref[pl.ds(j * _BS, _BS), :]
        lab = label_ref[pl.ds(j * _BS, _BS), :].astype(jnp.int32)
        onehot = (col_iota == lab).astype(jnp.float32)
        mean_g = jnp.sum(onehot * mean, axis=1, keepdims=True)
        d2 = (cos_f - mean_g) ** 2
        return sq + jnp.sum(onehot * d2, axis=0, keepdims=True)

    sq = jax.lax.fori_loop(0, _NB, phase2, zero_row)
    var = sq / jnp.maximum(counts - 1.0, 1.0)
    std = jnp.sqrt(jnp.maximum(var, 1e-12))
    inv_std = 1.0 / std

    def phase3(j, _):
        cos_f = cosf_ref[pl.ds(j * _BS, _BS), :]
        lab = label_ref[pl.ds(j * _BS, _BS), :].astype(jnp.int32)
        onehot = (col_iota == lab).astype(jnp.float32)
        mean_g = jnp.sum(onehot * mean, axis=1, keepdims=True)
        isd_g = jnp.sum(onehot * inv_std, axis=1, keepdims=True)
        z = (cos_f - mean_g) * isd_g
        pdf = jnp.exp(-0.5 * z * z) * isd_g * _INV_SQRT_2PI
        w = jnp.where(cos_f < mean_g, pdf, 1.0)
        weight_ref[pl.ds(j * _BS, _BS), :] = w
        return 0

    jax.lax.fori_loop(0, _NB, phase3, 0)


@functools.partial(jax.jit, static_argnames=())
def kernel(feature, pred, unlabeled_index, centroids):
    del pred, unlabeled_index
    cnt = jnp.zeros((_D, _KP), jnp.float32).at[:, :_K].set(centroids.T)
    label2d, weight2d = pl.pallas_call(
        _cluster_kernel,
        out_shape=(
            jax.ShapeDtypeStruct((_B, 1), jnp.float32),
            jax.ShapeDtypeStruct((_B, 1), jnp.float32),
        ),
        scratch_shapes=[pltpu.VMEM((_B, 1), jnp.float32)],
    )(feature, cnt)
    return label2d.reshape(_B), weight2d.reshape(_B)
